# baseline (speedup 1.0000x reference)
import functools

import jax
import jax.numpy as jnp
from jax import lax
from jax.experimental import pallas as pl
from jax.experimental.pallas import tpu as pltpu
from jax.experimental.pallas import tpu_sc as plsc

_DIM = 64
_C = 1024


def _dist_kernel(xt_ref, xsq_ref, e2_ref, esqb_ref, idx_ref):
    e2 = e2_ref[...]
    for r in range(xt_ref.shape[0]):
        xt = xt_ref[r]
        neg2cross = lax.dot_general(
            e2, xt, (((1,), (0,)), ((), ())), preferred_element_type=jnp.float32
        )
        x_sq = xsq_ref[r]
        n = xt.shape[1]
        subl = lax.broadcasted_iota(jnp.int32, (8, n), 0).astype(jnp.float32)
        run_s = jnp.full((8, n), jnp.inf, jnp.float32)
        run_i = jnp.zeros((8, n), jnp.float32)
        for j in range(_C // 8):
            sl = neg2cross[j * 8 : (j + 1) * 8, :]
            esl = esqb_ref[pl.ds(j * 8, 8), :]
            s = (x_sq + sl) + esl
            better = s < run_s
            run_s = jnp.where(better, s, run_s)
            run_i = jnp.where(better, subl + jnp.float32(j * 8), run_i)
        d = jnp.sqrt(jnp.clip(run_s, 0.0, None))
        dmin = jnp.min(d, axis=0, keepdims=True)
        idx = jnp.min(jnp.where(d == dmin, run_i, jnp.float32(_C)), axis=0)
        idx_ref[r, 0, :] = idx.astype(jnp.int32)


def _nearest_indices(x, e):
    b, n, dim = x.shape
    xt = jnp.transpose(x, (0, 2, 1))
    xsq = jnp.sum(x * x, axis=-1)[:, None, :]
    e2 = -2.0 * e
    esq = jnp.sum(e * e, axis=1)
    esqb = jnp.broadcast_to(esq[:, None], (_C, n))
    rb = 4
    idx3 = pl.pallas_call(
        _dist_kernel,
        grid=(b // rb,),
        in_specs=[
            pl.BlockSpec((rb, dim, n), lambda i: (i, 0, 0)),
            pl.BlockSpec((rb, 1, n), lambda i: (i, 0, 0)),
            pl.BlockSpec((_C, dim), lambda i: (0, 0)),
            pl.BlockSpec((_C, n), lambda i: (0, 0)),
        ],
        out_specs=pl.BlockSpec((rb, 1, n), lambda i: (i, 0, 0)),
        out_shape=jax.ShapeDtypeStruct((b, 1, n), jnp.int32),
        compiler_params=pltpu.CompilerParams(
            dimension_semantics=("parallel",)
        ),
    )(xt, xsq, e2, esqb)
    return idx3.reshape(b, n)


def _make_sc_gather(m):
    info = plsc.get_sparse_core_info()
    nc, ns = info.num_cores, info.num_subcores
    nw = nc * ns
    assert m % (8 * nw) == 0
    bpw = m // nw
    half = bpw // 2
    ch = 96
    assert half % ch == 0
    mesh = plsc.VectorSubcoreMesh(core_axis_name="c", subcore_axis_name="s")

    @functools.partial(
        pl.kernel,
        mesh=mesh,
        out_type=jax.ShapeDtypeStruct((m, 128), jnp.float32),
        scratch_types=[
            pltpu.VMEM((bpw,), jnp.int32),
            pltpu.VMEM((half, 128), jnp.float32),
            pltpu.SemaphoreType.DMA,
        ],
    )
    def gather(table_hbm, idx_hbm, out_hbm, idx_v, rows_v, sem):
        wid = lax.axis_index("s") * nc + lax.axis_index("c")
        base = wid * bpw
        pltpu.sync_copy(idx_hbm.at[pl.ds(base, bpw)], idx_v)
        for h in range(2):
            copies = [
                pltpu.async_copy(
                    table_hbm.at[idx_v.at[pl.ds(h * half + j * ch, ch)]],
                    rows_v.at[pl.ds(j * ch, ch)],
                    sem,
                )
                for j in range(half // ch)
            ]
            for c in copies:
                c.wait()
            pltpu.sync_copy(rows_v, out_hbm.at[pl.ds(base + h * half, half)])

    return gather


def _transpose_kernel(q_ref, out_ref):
    n = out_ref.shape[2]
    r0 = lax.broadcasted_iota(jnp.int32, (_DIM, _DIM), 0)
    r1 = lax.broadcasted_iota(jnp.int32, (_DIM, _DIM), 1)
    ident = jnp.where(r0 == r1, 1.0, 0.0).astype(jnp.float32)
    for r in range(out_ref.shape[0]):
        q = q_ref[pl.ds(r * n, n), : _DIM]
        qt = lax.dot_general(
            ident, q, (((1,), (1,)), ((), ())), preferred_element_type=jnp.float32
        )
        out_ref[r, :, :] = qt


def _transpose_rows(q128, b, n):
    rb = 4
    return pl.pallas_call(
        _transpose_kernel,
        grid=(b // rb,),
        in_specs=[pl.BlockSpec((rb * n, 128), lambda i: (i, 0))],
        out_specs=pl.BlockSpec((rb, _DIM, n), lambda i: (i, 0, 0)),
        out_shape=jax.ShapeDtypeStruct((b, _DIM, n), jnp.float32),
        compiler_params=pltpu.CompilerParams(
            dimension_semantics=("parallel",)
        ),
    )(q128)


def kernel(x, embed):
    b, n, d = x.shape
    e = embed[0]
    idx = _nearest_indices(x, e)
    e_pad = jnp.pad(e, ((0, 0), (0, 128 - _DIM)))
    q128 = _make_sc_gather(b * n)(e_pad, idx.reshape(b * n))
    qt3 = _transpose_rows(q128, b, n)
    return jnp.transpose(qt3, (0, 2, 1)), idx
